# baseline (speedup 1.0000x reference)
import functools

import jax
import jax.numpy as jnp
from jax import lax
from jax.experimental import pallas as pl
from jax.experimental.pallas import tpu as pltpu
from jax.experimental.pallas import tpu_sc as plsc

_B, _L, _D, _V = 1024, 200, 64, 100000
_NC, _NS = 2, 16
_NW = _NC * _NS
_LH = _L // 2
_NK = _D // 16

_NCHUNK = 4
_CB = _B // _NCHUNK
_BPW = _CB // _NW

_VB = 4096


def _make_bag_kernel(chunk):
    row0 = chunk * _CB

    def bag(x_hbm, emb_hbm, out_hbm, idx_v, rows_v, acc_v, sem):
        wid = lax.axis_index("s") * _NC + lax.axis_index("c")
        base = row0 + wid * _BPW
        pltpu.sync_copy(x_hbm.at[pl.ds(base, _BPW)], idx_v)

        def row_body(i, carry):
            cp0 = pltpu.async_copy(
                emb_hbm.at[idx_v.at[i, 0]], rows_v.at[pl.ds(0, _LH)], sem)
            cp1 = pltpu.async_copy(
                emb_hbm.at[idx_v.at[i, 1]], rows_v.at[pl.ds(_LH, _LH)], sem)
            cp0.wait()
            cp1.wait()

            def red(j, acc):
                return tuple(acc[k] + rows_v[j, pl.ds(16 * k, 16)]
                             for k in range(_NK))

            zeros = tuple(jnp.zeros((16,), jnp.float32) for _ in range(_NK))
            acc = lax.fori_loop(0, _L, red, zeros)
            for k in range(_NK):
                acc_v[i, pl.ds(16 * k, 16)] = acc[k]
            return carry

        lax.fori_loop(0, _BPW, row_body, 0)
        pltpu.sync_copy(acc_v, out_hbm.at[pl.ds(wid * _BPW, _BPW)])

    return bag


def _embedding_bag(x3, embedding_weight, chunk):
    mesh = plsc.VectorSubcoreMesh(core_axis_name="c", subcore_axis_name="s")
    k = functools.partial(
        pl.kernel,
        mesh=mesh,
        out_type=jax.ShapeDtypeStruct((_CB, _D), jnp.float32),
        scratch_types=[
            pltpu.VMEM((_BPW, 2, _LH), jnp.int32),
            pltpu.VMEM((_L, _D), jnp.float32),
            pltpu.VMEM((_BPW, _D), jnp.float32),
            pltpu.SemaphoreType.DMA,
        ],
        compiler_params=pltpu.CompilerParams(use_tc_tiling_on_sc=False),
    )(_make_bag_kernel(chunk))
    return k(x3, embedding_weight)


def _mm_first_kernel(x_ref, w_ref, b_ref, o_ref):
    o_ref[...] = lax.dot_general(
        x_ref[...], w_ref[...], (((1,), (1,)), ((), ())),
        preferred_element_type=jnp.float32) + b_ref[...]


def _mm_chain_kernel(x_ref, w_ref, b_ref, buf_ref, o_ref):
    o_ref[...] = lax.dot_general(
        x_ref[...], w_ref[...], (((1,), (1,)), ((), ())),
        preferred_element_type=jnp.float32) + b_ref[...]


def _matmul_chunk(x_sum_c, fc1_weight, bias2, chunk, buf):
    in_specs = [
        pl.BlockSpec((_CB, _D), lambda i: (0, 0)),
        pl.BlockSpec((_VB, _D), lambda i: (i, 0)),
        pl.BlockSpec((1, _VB), lambda i: (0, i)),
    ]
    args = [x_sum_c, fc1_weight, bias2]
    io_alias = {}
    body = _mm_first_kernel
    if buf is not None:
        in_specs.append(pl.BlockSpec(memory_space=pl.ANY))
        args.append(buf)
        io_alias = {3: 0}
        body = _mm_chain_kernel
    return pl.pallas_call(
        body,
        grid=(pl.cdiv(_V, _VB),),
        in_specs=in_specs,
        out_specs=pl.BlockSpec((_CB, _VB), lambda i: (chunk, i)),
        out_shape=jax.ShapeDtypeStruct((_B, _V), jnp.float32),
        input_output_aliases=io_alias,
    )(*args)


def kernel(x_in, embedding_weight, fc1_weight, fc1_bias):
    x3 = x_in.reshape(_B, 2, _LH)
    bias2 = fc1_bias.reshape(1, _V)
    x_sums = [_embedding_bag(x3, embedding_weight, c) for c in range(_NCHUNK)]
    buf = None
    for c in range(_NCHUNK):
        buf = _matmul_chunk(x_sums[c], fc1_weight, bias2, c, buf)
    return buf


# speedup vs baseline: 1.1631x; 1.1631x over previous
import functools

import jax
import jax.numpy as jnp
from jax import lax
from jax.experimental import pallas as pl
from jax.experimental.pallas import tpu as pltpu
from jax.experimental.pallas import tpu_sc as plsc

_B, _L, _D, _V = 1024, 200, 64, 100000
_NC, _NS = 2, 16
_NW = _NC * _NS
_BPW = _B // _NW
_LH = _L // 2
_NK = _D // 16

_VB = 4096


def _bag_kernel(x_hbm, emb_hbm, out_hbm, idx_v, rows_v, acc_v, sem0, sem1, sem2):
    wid = lax.axis_index("s") * _NC + lax.axis_index("c")
    base = wid * _BPW
    pltpu.sync_copy(x_hbm.at[pl.ds(base, _BPW)], idx_v)
    sems = (sem0, sem1)

    def issue(i, slot):
        return (
            pltpu.async_copy(emb_hbm.at[idx_v.at[i, 0]],
                             rows_v.at[slot, pl.ds(0, _LH)], sems[slot]),
            pltpu.async_copy(emb_hbm.at[idx_v.at[i, 1]],
                             rows_v.at[slot, pl.ds(_LH, _LH)], sems[slot]),
        )

    cps = [None, None]
    cps[0] = issue(0, 0)

    for i in range(_BPW):
        slot = i & 1
        if i + 1 < _BPW:
            cps[1 - slot] = issue(i + 1, 1 - slot)
        cps[slot][0].wait()
        cps[slot][1].wait()

        def red(t, acc):
            a = [acc[k] + rows_v[slot, 2 * t, pl.ds(16 * k, 16)]
                 for k in range(_NK)]
            b = [acc[_NK + k] + rows_v[slot, 2 * t + 1, pl.ds(16 * k, 16)]
                 for k in range(_NK)]
            return tuple(a + b)

        zeros = tuple(jnp.zeros((16,), jnp.float32) for _ in range(2 * _NK))
        acc = lax.fori_loop(0, _L // 2, red, zeros, unroll=4)
        for k in range(_NK):
            acc_v[i, pl.ds(16 * k, 16)] = acc[k] + acc[_NK + k]

    pltpu.async_copy(acc_v, out_hbm.at[pl.ds(base, _BPW)], sem2).wait()


def _embedding_bag(x3, embedding_weight):
    mesh = plsc.VectorSubcoreMesh(core_axis_name="c", subcore_axis_name="s")
    k = functools.partial(
        pl.kernel,
        mesh=mesh,
        out_type=jax.ShapeDtypeStruct((_B, _D), jnp.float32),
        scratch_types=[
            pltpu.VMEM((_BPW, 2, _LH), jnp.int32),
            pltpu.VMEM((2, _L, _D), jnp.float32),
            pltpu.VMEM((_BPW, _D), jnp.float32),
            pltpu.SemaphoreType.DMA,
            pltpu.SemaphoreType.DMA,
            pltpu.SemaphoreType.DMA,
        ],
        compiler_params=pltpu.CompilerParams(use_tc_tiling_on_sc=False),
    )(_bag_kernel)
    return k(x3, embedding_weight)


def _mm_kernel(x_ref, w_ref, b_ref, o_ref):
    o_ref[...] = lax.dot_general(
        x_ref[...], w_ref[...], (((1,), (1,)), ((), ())),
        preferred_element_type=jnp.float32) + b_ref[...]


def _matmul(x_sum, fc1_weight, fc1_bias):
    bias2 = fc1_bias.reshape(1, _V)
    return pl.pallas_call(
        _mm_kernel,
        grid=(pl.cdiv(_V, _VB),),
        in_specs=[
            pl.BlockSpec((_B, _D), lambda i: (0, 0)),
            pl.BlockSpec((_VB, _D), lambda i: (i, 0)),
            pl.BlockSpec((1, _VB), lambda i: (0, i)),
        ],
        out_specs=pl.BlockSpec((_B, _VB), lambda i: (0, i)),
        out_shape=jax.ShapeDtypeStruct((_B, _V), jnp.float32),
    )(x_sum, fc1_weight, bias2)


def kernel(x_in, embedding_weight, fc1_weight, fc1_bias):
    x3 = x_in.reshape(_B, 2, _LH)
    x_sum = _embedding_bag(x3, embedding_weight)
    return _matmul(x_sum, fc1_weight, fc1_bias)
